# Initial kernel scaffold; baseline (speedup 1.0000x reference)
import jax
import jax.numpy as jnp
from jax.experimental import pallas as pl


def kernel(q_head, q_rel, q_time, tail_nodes, tail_index, r_neighbor, t_neighbor, time_neighbor, hidden, tail_emd, batch_size, num_nodes, Wq, bq, Wa, ba, Watt, batt, Wrule, brule, Wout, bout):
    raise NotImplementedError("write your pallas kernel here")


# speedup vs baseline: 1.0122x; 1.0122x over previous
import functools

import jax
import jax.numpy as jnp
from jax.experimental import pallas as pl
from jax.experimental.pallas import tpu as pltpu

TOPK = 256


def _edge_body(q_ref, qh_ref, r_ref, t_ref, tm_ref, h_ref, wa_ref, ba_ref,
               watt_ref, batt_ref, wrule_ref, brule_ref, att_ref, msg_ref):
    r = r_ref[0]
    t = t_ref[0]
    tm = tm_ref[0]
    x = jnp.concatenate([r, t, tm], axis=-1)
    ne = jnp.dot(x, wa_ref[...]) + ba_ref[...]
    qb = q_ref[0]
    qa = jnp.concatenate([qb * ne, qb - ne, qb + ne], axis=-1)
    att1 = jax.nn.sigmoid(jnp.dot(qa, watt_ref[...]) + batt_ref[...])
    att2 = jax.nn.sigmoid(jnp.dot(h_ref[0], wrule_ref[...]) + brule_ref[...])
    att = (att1 + att2) / 2.0
    att_ref[0] = att
    msg_ref[0] = att * (qh_ref[0] + r + tm)


def _edge_pass(query_emd, q_head, r3, t3, tm3, hidden3, Wa, ba, Watt, batt, Wrule, brule):
    B, D = query_emd.shape
    N = r3.shape[1]
    full = lambda shape: pl.BlockSpec(shape, lambda b: tuple(0 for _ in shape))
    batch3 = pl.BlockSpec((1, N, D), lambda b: (b, 0, 0))
    att, msg = pl.pallas_call(
        _edge_body,
        grid=(B,),
        in_specs=[
            pl.BlockSpec((1, 1, D), lambda b: (b, 0, 0)),
            pl.BlockSpec((1, 1, D), lambda b: (b, 0, 0)),
            batch3, batch3, batch3, batch3,
            full((3 * D, D)),
            full((1, D)),
            full((3 * D, 1)),
            full((1, 1)),
            full((D, 1)),
            full((1, 1)),
        ],
        out_specs=[
            pl.BlockSpec((1, N, 1), lambda b: (b, 0, 0)),
            pl.BlockSpec((1, N, D), lambda b: (b, 0, 0)),
        ],
        out_shape=[
            jax.ShapeDtypeStruct((B, N, 1), jnp.float32),
            jax.ShapeDtypeStruct((B, N, D), jnp.float32),
        ],
    )(query_emd.reshape(B, 1, D), q_head.reshape(B, 1, D), r3, t3, tm3, hidden3,
      Wa.T, ba.reshape(1, D), Watt.T, batt.reshape(1, 1), Wrule.T, brule.reshape(1, 1))
    return att.reshape(B * N, 1), msg.reshape(B * N, D)


def kernel(q_head, q_rel, q_time, tail_nodes, tail_index, r_neighbor, t_neighbor, time_neighbor, hidden, tail_emd, batch_size, num_nodes, Wq, bq, Wa, ba, Watt, batt, Wrule, brule, Wout, bout):
    D = q_head.shape[-1]
    T = tail_nodes.shape[0]
    B = q_head.shape[0]
    N = r_neighbor.shape[1]
    size_zero = ((batch_size - B) + (num_nodes - N)).astype(jnp.float32) if hasattr(batch_size, "astype") else jnp.float32((batch_size - B) + (num_nodes - N))
    query_emd = jnp.concatenate([q_head, q_rel, q_time], axis=-1) @ Wq.T + bq
    att, message = _edge_pass(query_emd, q_head + size_zero, r_neighbor, t_neighbor,
                              time_neighbor, hidden.reshape(B, N, D),
                              Wa, ba, Watt, batt, Wrule, brule)
    tail_out = tail_emd + jax.ops.segment_sum(message, tail_index, num_segments=T)
    new_hidden = jax.ops.segment_sum(hidden, tail_index, num_segments=T)
    agg_att = jax.ops.segment_sum(att, tail_index, num_segments=T)[:, 0]
    nodes_batch = tail_nodes[:, 0]
    nodes_tail = tail_nodes[:, 1]
    nodes_time = tail_nodes[:, 2]

    def select_batch(i):
        mask = nodes_batch == i
        count = jnp.sum(mask)
        order = jnp.argsort(jnp.where(mask, 0, 1), stable=True)[:TOPK]
        valid = jnp.arange(TOPK) < count
        pad_tail = jnp.where(valid, nodes_tail[order], -jnp.ones((), nodes_tail.dtype))
        pad_time = jnp.where(valid, nodes_time[order], jnp.zeros((), nodes_time.dtype))
        pad_emd = jnp.where(valid[:, None], tail_out[order], jnp.zeros((), tail_out.dtype))
        pad_hid = jnp.where(valid[:, None], new_hidden[order], jnp.zeros((), new_hidden.dtype))
        idx = jax.lax.top_k(jnp.where(mask, agg_att, -jnp.inf), TOPK)[1]
        use_tk = count >= TOPK
        temp_tail = jnp.where(use_tk, nodes_tail[idx], pad_tail)
        temp_time = jnp.where(use_tk, nodes_time[idx], pad_time)
        temp_emd = jnp.where(use_tk, tail_out[idx], pad_emd)
        hid = jnp.where(use_tk, new_hidden[idx], pad_hid)
        return temp_tail, temp_time, temp_emd, hid

    tail_stack, time_stack, emd_stack, hidden_stack = jax.vmap(select_batch)(jnp.arange(B))
    new_nodes = jnp.stack([tail_stack, time_stack], axis=-1)
    tail_final = emd_stack @ Wout.T + bout
    return (new_nodes, tail_final, hidden_stack)
